# baseline (speedup 1.0000x reference)
import functools

import jax
import jax.numpy as jnp
from jax import lax
from jax.experimental import pallas as pl
from jax.experimental.pallas import tpu as pltpu
from jax.experimental.pallas import tpu_sc as plsc

_NEG = -1e30


def _leaky(v):
    return jnp.where(v > 0, v, 0.2 * v)


def _dot(a, b, dims):
    return lax.dot_general(a, b, (dims, ((), ())),
                           preferred_element_type=jnp.float32,
                           precision=lax.Precision.HIGHEST)


def _mm(a, b):
    return _dot(a, b, ((1,), (0,)))


def _seg_softmax_ne(logit_row, gdst_b):
    s = jnp.where(gdst_b, logit_row, _NEG)
    m = jnp.max(s, axis=1, keepdims=True)
    ex = jnp.where(gdst_b, jnp.exp(s - m), 0.0)
    den = jnp.sum(ex, axis=1, keepdims=True) + 1e-16
    return ex / den


_SC_INFO = plsc.get_sparse_core_info()
_SC_NC = _SC_INFO.num_cores


def _bfly(v, op):
    lane = lax.iota(jnp.int32, 16)
    for k in (1, 2, 4, 8):
        v = op(v, v.at[lane ^ k].get(mode="promise_in_bounds"))
    return v


def _rsum(v):
    return _bfly(v, jnp.add)


def _rmax(v):
    return _bfly(v, jnp.maximum)


def _make_sc_edge():
    mesh = plsc.VectorSubcoreMesh(core_axis_name="c", subcore_axis_name="s")
    f32 = jnp.float32

    @functools.partial(
        pl.kernel, mesh=mesh,
        out_type=jax.ShapeDtypeStruct((32,), f32),
        scratch_types=[
            pltpu.VMEM((40,), jnp.int32),
            pltpu.VMEM((40, 128), f32),
            pltpu.VMEM((32,), f32),
            pltpu.VMEM((32,), jnp.int32),
            pltpu.VMEM((32,), f32),
            pltpu.VMEM((32,), f32),
            pltpu.SemaphoreType.DMA,
        ])
    def sc_edge(table_hbm, idx_hbm, att_hbm, dst_hbm, ea_hbm, out_hbm,
                idx_v, rows_v, att_v, dst_v, ea_v, ew_v, sem):
        wid = lax.axis_index("s") * _SC_NC + lax.axis_index("c")

        pltpu.sync_copy(idx_hbm, idx_v)
        pltpu.sync_copy(att_hbm, att_v)
        pltpu.sync_copy(dst_hbm, dst_v)
        pltpu.sync_copy(ea_hbm, ea_v)
        pltpu.async_copy(table_hbm.at[idx_v], rows_v, sem).wait()

        lane = lax.iota(jnp.int32, 16)
        att_a = att_v[0:16]
        att_b = att_v[16:32]
        l0 = jnp.zeros((16,), f32)
        l1 = jnp.zeros((16,), f32)
        for e in range(20):
            a = rows_v[e, 0:16] + rows_v[20 + e, 0:16]
            b = rows_v[e, 16:32] + rows_v[20 + e, 16:32]
            a = jnp.where(a > 0, a, 0.2 * a)
            b = jnp.where(b > 0, b, 0.2 * b)
            s = _rsum(att_a * a + att_b * b)
            if e < 16:
                l0 = l0 + jnp.where(lane == e, s, 0.0)
            else:
                l1 = l1 + jnp.where(lane == (e - 16), s, 0.0)

        d0 = dst_v[0:16]
        d1 = dst_v[16:32]
        a0 = jnp.zeros((16,), f32)
        a1 = jnp.zeros((16,), f32)
        for n in range(5):
            m0 = d0 == n
            m1 = d1 == n
            s0 = jnp.where(m0, l0, _NEG)
            s1 = jnp.where(m1, l1, _NEG)
            mx = jnp.maximum(_rmax(s0), _rmax(s1))
            e0 = jnp.where(m0, jnp.exp(s0 - mx), 0.0)
            e1 = jnp.where(m1, jnp.exp(s1 - mx), 0.0)
            den = _rsum(e0) + _rsum(e1) + 1e-16
            a0 = a0 + e0 / den
            a1 = a1 + e1 / den

        ew_v[0:16] = a0 * ea_v[0:16]
        ew_v[16:32] = a1 * ea_v[16:32]

        @pl.when(wid == 0)
        def _():
            pltpu.sync_copy(ew_v, out_hbm)

    return sc_edge


def kernel(x, edge_index, edge_attr, params):
    p = params
    N = x.shape[0]
    E = edge_index.shape[1]
    CT_N = p["ca_W3"].shape[1]
    H = p["lstm_Whh"].shape[1]
    OUT = p["gin_W2_3"].shape[1]
    f32 = jnp.float32

    vecs = [p["lstm_bih"], p["lstm_bhh"],
            p["gin_b1_0"], p["gin_b1_1"], p["gin_b1_2"],
            p["gin_b2_0"], p["gin_b2_1"], p["gin_b2_2"],
            p["ca_asrc0"], p["ca_adst0"], p["ca_b0"],
            p["ca_asrc1"], p["ca_adst1"], p["ca_b1"],
            p["ca_asrc2"], p["ca_adst2"], p["ca_b2"],
            p["g2_att"],
            p["ca_asrc3"], p["ca_adst3"], p["ca_b3"],
            p["ct_asrc1"], p["ct_adst1"], p["ct_b1"],
            p["ct_asrc0"], p["ct_adst0"], p["ct_b0"],
            p["gin_b1_3"], jnp.zeros((6,), f32), p["gin_b2_3"]]
    off, voff = 0, []
    for v in vecs:
        voff.append(off)
        off += v.shape[0]
    vecpack = jnp.concatenate(vecs)[None, :]

    pack10 = jnp.concatenate(
        [p["gin_W1_3"], p["gin_W2_3"], jnp.zeros((6, OUT), f32)], axis=0)
    pack32 = jnp.concatenate(
        [p["g2_Wl"], jnp.zeros((6, CT_N), f32), p["g2_Wr"],
         jnp.zeros((6, CT_N), f32), p["ca_W3"], p["ct_W1"]], axis=0)
    pack64 = jnp.concatenate([p["ca_W0"], p["ca_W1"], p["ca_W2"]], axis=0)
    pack256 = jnp.concatenate(
        [p["gin_W1_0"], p["gin_W1_1"], p["gin_W1_2"],
         p["gin_W2_0"], p["gin_W2_1"], p["gin_W2_2"]], axis=0)

    def vec_of(ref, i, n):
        return ref[0:1, voff[i]: voff[i] + n]

    def pre(x_ref, w_ref, vp_ref, p32_ref, table_ref, h_ref):
        xv = x_ref[...]
        gates = (_dot(xv, w_ref[...], ((1,), (1,)))
                 + vec_of(vp_ref, 0, 4 * H) + vec_of(vp_ref, 1, 4 * H))
        c = jax.nn.sigmoid(gates[:, 0:H]) * jnp.tanh(gates[:, 2 * H:3 * H])
        h_ref[...] = jax.nn.sigmoid(gates[:, 3 * H:4 * H]) * jnp.tanh(c)
        table_ref[...] = jnp.zeros((16, 128), f32)
        table_ref[0:5, 0:CT_N] = _mm(xv, p32_ref[0:10, :])
        table_ref[8:13, 0:CT_N] = _mm(xv, p32_ref[16:26, :])

    table, h = pl.pallas_call(
        pre,
        out_shape=[jax.ShapeDtypeStruct((16, 128), f32),
                   jax.ShapeDtypeStruct((N, H), f32)],
    )(x, p["lstm_Wih"], vecpack, pack32)

    src = edge_index[0].astype(jnp.int32)
    dst = edge_index[1].astype(jnp.int32)
    idx40 = jnp.concatenate([src, dst + 8])
    dstpad = jnp.concatenate([dst, jnp.full((32 - E,), 7, jnp.int32)])
    eapad = jnp.concatenate([edge_attr[:, 0], jnp.zeros((32 - E,), f32)])
    ew32 = _make_sc_edge()(table, idx40, p["g2_att"], dstpad, eapad)

    args = [edge_index, ew32[None, :], vecpack, h,
            pack10, p["ct_W0"], pack32, pack64, pack256]

    def fused(ei_ref, ew_ref, vp_ref, h_ref, p10_ref, w16_ref, p32_ref,
              p64_ref, p256_ref, out_ref):
        ei = ei_ref[...]
        ew_row = ew_ref[0:1, 0:E]

        def vec(i, n):
            return vec_of(vp_ref, i, n)

        src_row = ei[0:1, :]
        dst_row = ei[1:2, :]

        node5 = lax.broadcasted_iota(jnp.int32, (N, E), 0)
        node32 = lax.broadcasted_iota(jnp.int32, (CT_N, E), 0)
        gs5_b = node5 == src_row
        gd5_b = node5 == dst_row
        gs5 = gs5_b.astype(f32)
        gd5 = gd5_b.astype(f32)
        gs32 = (node32 == src_row).astype(f32)
        gd32 = (node32 == dst_row).astype(f32)
        ri = lax.broadcasted_iota(jnp.int32, (CT_N, CT_N), 0)
        ci = lax.broadcasted_iota(jnp.int32, (CT_N, CT_N), 1)
        offdiag = ri != ci
        eye32 = (ri == ci).astype(f32)

        ca_w = [p64_ref[0:256, :], p64_ref[256:320, :],
                p64_ref[320:384, :], p32_ref[32:96, :]]
        ca_as, ca_ad, ca_b = [8, 11, 14, 18], [9, 12, 15, 19], [10, 13, 16, 20]
        z = h_ref[...]
        for li in range(4):
            fdim = ca_w[li].shape[1]
            hw = _mm(z, ca_w[li])
            s_src = jnp.sum(hw * vec(ca_as[li], fdim), axis=1, keepdims=True)
            s_dst = jnp.sum(hw * vec(ca_ad[li], fdim), axis=1, keepdims=True)
            e_row = _leaky(jnp.sum(gs5 * s_src, axis=0, keepdims=True)
                           + jnp.sum(gd5 * s_dst, axis=0, keepdims=True))
            alpha = _seg_softmax_ne(e_row, gd5_b) * ew_row
            msgs = _dot(gs5, hw, ((0,), (0,)))
            if li < 3:
                z = jnp.maximum(_mm(alpha, msgs) + vec(ca_b[li], fdim), 0.0)
            else:
                b_col = _dot(eye32, vec(ca_b[li], fdim), ((1,), (1,)))
                z = _dot(msgs, alpha, ((0,), (1,))) + b_col

        t = z
        ct_w = [w16_ref[...], p32_ref[96:112, :]]
        ct_as, ct_ad, ct_b = [24, 21], [25, 22], [26, 23]
        for li in range(2):
            w = ct_w[li]
            fdim = w.shape[1]
            hw = _mm(t, w)
            hwT = _dot(w, t, ((0,), (1,)))
            s_src_row = _dot(vec(ct_as[li], fdim), hwT, ((1,), (0,)))
            s_dst_col = jnp.sum(hw * vec(ct_ad[li], fdim), axis=1,
                                keepdims=True)
            ematT = _leaky(s_dst_col + s_src_row)
            ematT = jnp.where(offdiag, ematT, _NEG)
            m = jnp.max(ematT, axis=1, keepdims=True)
            ex = jnp.where(offdiag, jnp.exp(ematT - m), 0.0)
            alphaT = ex / (jnp.sum(ex, axis=1, keepdims=True) + 1e-16)
            if li < 1:
                t = jnp.maximum(_mm(alphaT, hw) + vec(ct_b[li], fdim), 0.0)
            else:
                b_col = _dot(eye32, vec(ct_b[li], fdim), ((1,), (1,)))
                y = _dot(hw, alphaT, ((0,), (1,))) + b_col

        adj = _dot(gd32, gs32, ((1,), (1,)))
        gin_w1 = [p256_ref[0:32, :], p256_ref[32:288, :],
                  p256_ref[288:544, :], p10_ref[0:256, :]]
        gin_w2 = [p256_ref[544:800, :], p256_ref[800:1056, :],
                  p256_ref[1056:1312, :], p10_ref[256:266, :]]
        for li in range(4):
            b1 = vec(2 + li, H) if li < 3 else vec(27, OUT)
            b2 = vec(5 + li, H) if li < 3 else vec(29, OUT)
            hg = y + _mm(adj, y)
            hg = jnp.maximum(_mm(hg, gin_w1[li]) + b1, 0.0)
            y = _mm(hg, gin_w2[li]) + b2
            if li < 3:
                y = jnp.maximum(y, 0.0)

        out_ref[...] = y

    return pl.pallas_call(
        fused,
        out_shape=jax.ShapeDtypeStruct((CT_N, OUT), jnp.float32),
    )(*args)
